# baseline (speedup 1.0000x reference)
import functools

import jax
import jax.numpy as jnp
from jax import lax
from jax.experimental import pallas as pl
from jax.experimental.pallas import tpu as pltpu
from jax.experimental.pallas import tpu_sc as plsc

D_MODEL = 64
SCALE = 8.0
NUM_CORES = 2
NUM_SUBCORES = 16
NUM_WORKERS = NUM_CORES * NUM_SUBCORES
LANES = 16
CHUNK = 200
N_T = 200
K_T = N_T // 2


def _emb_call(n_idx):
    b_per_w = n_idx // NUM_WORKERS
    rows_per_w = b_per_w // N_T
    steps = b_per_w // CHUNK
    groups = steps // 2
    n_b = n_idx // N_T
    icap = ((CHUNK + LANES - 1) // LANES) * LANES
    full_g = CHUNK // LANES
    tail = CHUNK - full_g * LANES
    mesh = plsc.VectorSubcoreMesh(
        core_axis_name="c", subcore_axis_name="s",
        num_cores=NUM_CORES, num_subcores=NUM_SUBCORES)

    @functools.partial(
        pl.kernel,
        out_type=jax.ShapeDtypeStruct((K_T, n_b, 2 * D_MODEL),
                                      jnp.float32),
        mesh=mesh,
        compiler_params=pltpu.CompilerParams(
            use_tc_tiling_on_sc=True, needs_layout_passes=False),
        scratch_types=[
            pltpu.VMEM((icap,), jnp.int32),
            pltpu.VMEM((icap,), jnp.int32),
            pltpu.VMEM((icap,), jnp.int32),
            pltpu.VMEM((icap,), jnp.int32),
            pltpu.VMEM((CHUNK, 2 * D_MODEL), jnp.float32),
            pltpu.VMEM((CHUNK, 2 * D_MODEL), jnp.float32),
            pltpu.VMEM((K_T, 2 * D_MODEL), jnp.float32),
            pltpu.VMEM((K_T, 2 * D_MODEL), jnp.float32),
            pltpu.SemaphoreType.DMA,
            pltpu.SemaphoreType.DMA,
            pltpu.SemaphoreType.DMA,
            pltpu.SemaphoreType.DMA,
        ],
    )
    def emb(idx_hbm, table_hbm, out_hbm, ic0, ic1, rid0, rid1, rows0, rows1,
            ob0, ob1, gsem0, gsem1, osem0, osem1):
        wid = lax.axis_index("s") * NUM_CORES + lax.axis_index("c")
        base = wid * b_per_w
        b0 = wid * rows_per_w
        ics = (ic0, ic1)
        rids = (rid0, rid1)
        rows = (rows0, rows1)
        obufs = (ob0, ob1)
        gsems = (gsem0, gsem1)
        osems = (osem0, osem1)

        def prep_rids(g, b):
            pltpu.sync_copy(idx_hbm.at[pl.ds(base + g * CHUNK, CHUNK)],
                            ics[b].at[pl.ds(0, CHUNK)])

            def body(k, _):
                sl = pl.ds(k * LANES, LANES)
                iv = ics[b][sl]
                rids[b][sl] = jnp.where(iv < H_SPLIT, iv, iv - H_SPLIT)
                return 0
            lax.fori_loop(0, (icap // LANES), body, 0)

        def gather(b):
            return pltpu.make_async_copy(
                table_hbm.at[rids[b].at[pl.ds(0, CHUNK)]],
                rows[b], gsems[b])

        def writeout(g, b):
            return pltpu.make_async_copy(
                obufs[b], out_hbm.at[:, b0 + g], osems[b])

        prep_rids(0, 0)
        gather(0).start()

        def select_scale(g, b):
            buf = rows[b]
            ob = obufs[b]

            def do_group(k, nlanes):
                iv_vec = ics[b][pl.ds(k * LANES, LANES)]
                hv = jnp.where(iv_vec < H_SPLIT, 0, D_MODEL)
                for rl in range(nlanes):
                    h64 = hv[rl]
                    r = k * LANES + rl
                    dr = k * (LANES // 2) + rl // 2
                    db = (rl & 1) * D_MODEL
                    for j in range(D_MODEL // LANES):
                        src = buf[r, pl.ds(h64 + j * LANES, LANES)]
                        ob[dr, pl.ds(db + j * LANES, LANES)] = src

            @plsc.parallel_loop(0, full_g, step=1)
            def _(k):
                do_group(k, LANES)

            do_group(full_g, tail)

        def group(q, _):
            for b in (0, 1):
                g = q * 2 + b
                gather(b).wait()

                @pl.when(g >= 1)
                def _():
                    writeout(g - 1, 1 - b).wait()

                @pl.when(g + 1 < steps)
                def _():
                    prep_rids(g + 1, 1 - b)
                    gather(1 - b).start()

                select_scale(g, b)
                writeout(g, b).start()
            return 0

        lax.fori_loop(0, groups, group, 0)
        writeout(steps - 1, 1).wait()

    return emb


_PACK_W = 128
H_SPLIT = 512000


def _eye(n):
    r = lax.broadcasted_iota(jnp.int32, (n, n), 0)
    c = lax.broadcasted_iota(jnp.int32, (n, n), 1)
    return (r == c).astype(jnp.float32)


def _pack_table(table_t):
    grid = H_SPLIT // _PACK_W

    def body(left_ref, right_ref, out_ref):
        eye = _eye(D_MODEL)
        dn = (((0,), (0,)), ((), ()))
        out_ref[:, 0:D_MODEL] = lax.dot_general(
            left_ref[...] * SCALE, eye, dn,
            preferred_element_type=jnp.float32)
        out_ref[:, D_MODEL:2 * D_MODEL] = lax.dot_general(
            right_ref[...] * SCALE, eye, dn,
            preferred_element_type=jnp.float32)

    return pl.pallas_call(
        body,
        grid=(grid,),
        in_specs=[
            pl.BlockSpec((D_MODEL, _PACK_W), lambda c: (0, c)),
            pl.BlockSpec((D_MODEL, _PACK_W),
                         lambda c: (0, c + H_SPLIT // _PACK_W)),
        ],
        out_specs=pl.BlockSpec((_PACK_W, 2 * D_MODEL), lambda c: (c, 0)),
        out_shape=jax.ShapeDtypeStruct((H_SPLIT, 2 * D_MODEL), jnp.float32),
    )(table_t, table_t)


_BBLK = 128


def _repack_out(out2t, n_b):
    def body(in_ref, out_ref):
        eye = _eye(_BBLK)
        dn = (((0,), (0,)), ((), ()))
        for k in range(K_T):
            for s in range(2):
                blk = in_ref[k, :, pl.ds(s * D_MODEL, D_MODEL)]
                out_ref[k, s, :, :] = lax.dot_general(
                    blk, eye, dn, preferred_element_type=jnp.float32)

    return pl.pallas_call(
        body,
        grid=(n_b // _BBLK,),
        in_specs=[pl.BlockSpec((K_T, _BBLK, 2 * D_MODEL),
                               lambda c: (0, c, 0))],
        out_specs=pl.BlockSpec((K_T, 2, D_MODEL, _BBLK),
                               lambda c: (0, 0, 0, c)),
        out_shape=jax.ShapeDtypeStruct((K_T, 2, D_MODEL, n_b), jnp.float32),
    )(out2t)


def kernel(x, table):
    n_b, n_t = x.shape
    n_idx = x.size
    idx = x.reshape(n_idx).astype(jnp.int32)
    t2 = _pack_table(table.T)
    out2t = _emb_call(n_idx)(idx, t2)
    out3 = _repack_out(out2t, n_b)
    return out3.reshape(n_t, D_MODEL, n_b).transpose(2, 0, 1)


# speedup vs baseline: 2.6181x; 2.6181x over previous
import functools

import jax
import jax.numpy as jnp
from jax import lax
from jax.experimental import pallas as pl
from jax.experimental.pallas import tpu as pltpu
from jax.experimental.pallas import tpu_sc as plsc

D_MODEL = 64
SCALE = 8.0
NUM_CORES = 2
NUM_SUBCORES = 16
NUM_WORKERS = NUM_CORES * NUM_SUBCORES
LANES = 16
CHUNK = 200
N_T = 200
K_T = N_T // 2


def _emb_call(n_idx):
    b_per_w = n_idx // NUM_WORKERS
    rows_per_w = b_per_w // N_T
    steps = b_per_w // CHUNK
    groups = steps // 2
    n_b = n_idx // N_T
    icap = ((CHUNK + LANES - 1) // LANES) * LANES
    full_g = CHUNK // LANES
    tail = CHUNK - full_g * LANES
    mesh = plsc.VectorSubcoreMesh(
        core_axis_name="c", subcore_axis_name="s",
        num_cores=NUM_CORES, num_subcores=NUM_SUBCORES)

    @functools.partial(
        pl.kernel,
        out_type=jax.ShapeDtypeStruct((K_T, n_b, 2 * D_MODEL),
                                      jnp.float32),
        mesh=mesh,
        compiler_params=pltpu.CompilerParams(
            use_tc_tiling_on_sc=True, needs_layout_passes=False),
        scratch_types=[
            pltpu.VMEM((icap,), jnp.int32),
            pltpu.VMEM((icap,), jnp.int32),
            pltpu.VMEM((icap,), jnp.int32),
            pltpu.VMEM((icap,), jnp.int32),
            pltpu.VMEM((CHUNK, 2 * D_MODEL), jnp.float32),
            pltpu.VMEM((CHUNK, 2 * D_MODEL), jnp.float32),
            pltpu.VMEM((K_T, 2 * D_MODEL), jnp.float32),
            pltpu.VMEM((K_T, 2 * D_MODEL), jnp.float32),
            pltpu.SemaphoreType.DMA,
            pltpu.SemaphoreType.DMA,
            pltpu.SemaphoreType.DMA,
            pltpu.SemaphoreType.DMA,
        ],
    )
    def emb(idx_hbm, table_hbm, out_hbm, ic0, ic1, rid0, rid1, rows0, rows1,
            ob0, ob1, gsem0, gsem1, osem0, osem1):
        wid = lax.axis_index("s") * NUM_CORES + lax.axis_index("c")
        base = wid * b_per_w
        b0 = wid * rows_per_w
        ics = (ic0, ic1)
        rids = (rid0, rid1)
        rows = (rows0, rows1)
        obufs = (ob0, ob1)
        gsems = (gsem0, gsem1)
        osems = (osem0, osem1)

        def prep_rids(g, b):
            pltpu.sync_copy(idx_hbm.at[pl.ds(base + g * CHUNK, CHUNK)],
                            ics[b].at[pl.ds(0, CHUNK)])

            def body(k, _):
                sl = pl.ds(k * LANES, LANES)
                iv = ics[b][sl]
                rids[b][sl] = lax.shift_right_logical(iv, 1)
                return 0
            lax.fori_loop(0, (icap // LANES), body, 0)

        def gather(b):
            return pltpu.make_async_copy(
                table_hbm.at[rids[b].at[pl.ds(0, CHUNK)]],
                rows[b], gsems[b])

        def writeout(g, b):
            return pltpu.make_async_copy(
                obufs[b], out_hbm.at[:, b0 + g], osems[b])

        prep_rids(0, 0)
        gather(0).start()

        def select_scale(g, b):
            buf = rows[b]
            ob = obufs[b]

            def do_group(k, nlanes):
                iv_vec = ics[b][pl.ds(k * LANES, LANES)]
                hv = (iv_vec & 1) * D_MODEL
                for rl in range(nlanes):
                    h64 = hv[rl]
                    r = k * LANES + rl
                    dr = k * (LANES // 2) + rl // 2
                    db = (rl & 1) * D_MODEL
                    for j in range(D_MODEL // LANES):
                        src = buf[r, pl.ds(h64 + j * LANES, LANES)]
                        ob[dr, pl.ds(db + j * LANES, LANES)] = src * SCALE

            @plsc.parallel_loop(0, full_g, step=1)
            def _(k):
                do_group(k, LANES)

            do_group(full_g, tail)

        def group(q, _):
            for b in (0, 1):
                g = q * 2 + b
                gather(b).wait()

                @pl.when(g >= 1)
                def _():
                    writeout(g - 1, 1 - b).wait()

                @pl.when(g + 1 < steps)
                def _():
                    prep_rids(g + 1, 1 - b)
                    gather(1 - b).start()

                select_scale(g, b)
                writeout(g, b).start()
            return 0

        lax.fori_loop(0, groups, group, 0)
        writeout(steps - 1, 1).wait()

    return emb


_PACK_W = 128
H_SPLIT = 512000


def _eye(n):
    r = lax.broadcasted_iota(jnp.int32, (n, n), 0)
    c = lax.broadcasted_iota(jnp.int32, (n, n), 1)
    return (r == c).astype(jnp.float32)


def _pack_table(table_t):
    grid = H_SPLIT // _PACK_W

    def body(left_ref, right_ref, out_ref):
        eye = _eye(D_MODEL)
        dn = (((0,), (0,)), ((), ()))
        out_ref[:, 0:D_MODEL] = lax.dot_general(
            left_ref[...] * SCALE, eye, dn,

            preferred_element_type=jnp.float32)
        out_ref[:, D_MODEL:2 * D_MODEL] = lax.dot_general(
            right_ref[...] * SCALE, eye, dn,

            preferred_element_type=jnp.float32)

    return pl.pallas_call(
        body,
        grid=(grid,),
        in_specs=[
            pl.BlockSpec((D_MODEL, _PACK_W), lambda c: (0, c)),
            pl.BlockSpec((D_MODEL, _PACK_W),
                         lambda c: (0, c + H_SPLIT // _PACK_W)),
        ],
        out_specs=pl.BlockSpec((_PACK_W, 2 * D_MODEL), lambda c: (c, 0)),
        out_shape=jax.ShapeDtypeStruct((H_SPLIT, 2 * D_MODEL), jnp.float32),
    )(table_t, table_t)


_BBLK = 128


def _repack_out(out2t, n_b):
    def body(in_ref, out_ref):
        eye = _eye(_BBLK)
        dn = (((0,), (0,)), ((), ()))
        for k in range(K_T):
            for s in range(2):
                blk = in_ref[k, :, pl.ds(s * D_MODEL, D_MODEL)]
                out_ref[k, s, :, :] = lax.dot_general(
                    blk, eye, dn,
                    preferred_element_type=jnp.float32)

    return pl.pallas_call(
        body,
        grid=(n_b // _BBLK,),
        in_specs=[pl.BlockSpec((K_T, _BBLK, 2 * D_MODEL),
                               lambda c: (0, c, 0))],
        out_specs=pl.BlockSpec((K_T, 2, D_MODEL, _BBLK),
                               lambda c: (0, 0, 0, c)),
        out_shape=jax.ShapeDtypeStruct((K_T, 2, D_MODEL, n_b), jnp.float32),
    )(out2t)


def kernel(x, table):
    n_b, n_t = x.shape
    n_idx = x.size
    idx = x.reshape(n_idx).astype(jnp.int32)
    t2 = table.reshape(table.shape[0] // 2, 2 * D_MODEL)
    out2t = _emb_call(n_idx)(idx, t2)
    out3 = _repack_out(out2t, n_b)
    return out3.reshape(n_t, D_MODEL, n_b).transpose(2, 0, 1)
